# baseline (speedup 1.0000x reference)
import functools

import jax
import jax.numpy as jnp
from jax.experimental import pallas as pl
from jax.experimental.pallas import tpu as pltpu

_N = 2048
_EPS = 1e-8
_NCORES = 2


def _dnc_step(x_ref, wx_ref, wr_ref, whh_ref, bl_ref, wpost_ref, bpost_ref,
              y_ref, sl_ref,
              mem_ref, h_ref, c_ref, rrow_ref, sq_ref, slacc_ref,
              krow_ref, scal_ref,
              *, T, H, OUT, C):
    t = pl.program_id(1)

    @pl.when(t == 0)
    def _():
        mem_ref[...] = jnp.zeros_like(mem_ref)
        h_ref[...] = jnp.zeros_like(h_ref)
        c_ref[...] = jnp.zeros_like(c_ref)
        rrow_ref[...] = jnp.zeros_like(rrow_ref)
        sq_ref[...] = jnp.zeros_like(sq_ref)
        slacc_ref[...] = jnp.zeros_like(slacc_ref)

    x = x_ref[0]
    h = h_ref[...]
    c = c_ref[...]
    BC = h.shape[0]
    r = rrow_ref[...].reshape(BC, C)

    gates = (jnp.dot(x, wx_ref[...], preferred_element_type=jnp.float32)
             + jnp.dot(r, wr_ref[...], preferred_element_type=jnp.float32)
             + jnp.dot(h, whh_ref[...], preferred_element_type=jnp.float32)
             + bl_ref[...])
    i_g = gates[:, :H]
    f_g = gates[:, H:2 * H]
    g_g = gates[:, 2 * H:3 * H]
    o_g = gates[:, 3 * H:]
    c_new = jax.nn.sigmoid(f_g) * c + jax.nn.sigmoid(i_g) * jnp.tanh(g_g)
    h_new = jax.nn.sigmoid(o_g) * jnp.tanh(c_new)
    h_ref[...] = h_new
    c_ref[...] = c_new

    post = (jnp.dot(h_new, wpost_ref[...], preferred_element_type=jnp.float32)
            + bpost_ref[...])
    y_ref[0] = post[:, :OUT]
    key = post[:, OUT:OUT + C]
    gen = post[:, OUT + C:OUT + 2 * C]
    beta = jax.nn.softplus(post[:, OUT + 2 * C:OUT + 2 * C + 1])

    slacc_ref[...] += (key - gen) ** 2

    kk = jnp.sum(key * key, axis=1, keepdims=True)
    knorm = jnp.maximum(jnp.sqrt(kk), _EPS)

    krow_ref[...] = key.reshape(BC, 1, C)
    scal = jnp.pad(jnp.concatenate([beta, knorm, kk], axis=1),
                   ((0, 0), (0, 125)))
    scal_ref[...] = scal.reshape(BC, 1, 128)

    G = 4

    def group(g, carry):
        for j in range(G):
            b = g * G + j
            kc = krow_ref[b].T
            sc = scal_ref[b]
            beta_b = sc[:, 0:1]
            knorm_b = sc[:, 1:2]
            kk_b = sc[:, 2:3]
            slab = mem_ref[b]
            num_b = jnp.sum(slab * kc, axis=0, keepdims=True)
            sq_b = sq_ref[b]
            mnorm = jnp.maximum(jnp.sqrt(jnp.maximum(sq_b, 0.0)), _EPS)
            a = beta_b * (num_b / (mnorm * knorm_b))
            e = jnp.exp(a - jnp.max(a, axis=1, keepdims=True))
            w_b = e / jnp.sum(e, axis=1, keepdims=True)
            upd = slab + kc * w_b
            mem_ref[b] = upd
            rrow_ref[b] = jnp.sum(upd * w_b, axis=1, keepdims=True).T
            sq_ref[b] = sq_b + 2.0 * w_b * num_b + (w_b * w_b) * kk_b
        return carry

    jax.lax.fori_loop(0, BC // G, group, 0)

    @pl.when(t == T - 1)
    def _():
        sl_ref[0] = jnp.full((8, 128), jnp.sum(slacc_ref[...]), jnp.float32)


def kernel(xs, W_ih, W_hh, b_lstm, W_out, b_out, W_key, b_key,
           W_beta, b_beta, W_gen, b_gen):
    T, B, IN = xs.shape
    H = W_hh.shape[0]
    OUT = W_out.shape[1]
    C = W_key.shape[1]
    R = (W_ih.shape[0] - IN) // C
    N = _N
    BC = B // _NCORES

    W_x = W_ih[:IN]
    W_r = W_ih[IN:].reshape(C, R, 4 * H).sum(axis=1)
    post_w = jnp.concatenate([W_out, W_key, W_gen, W_beta], axis=1)
    post_pad = (-post_w.shape[1]) % 128
    W_post = jnp.pad(post_w, ((0, 0), (0, post_pad)))
    b_post = jnp.pad(jnp.concatenate([b_out, b_key, b_gen, b_beta]),
                     (0, post_pad))[None]
    bl = b_lstm[None]

    body = functools.partial(_dnc_step, T=T, H=H, OUT=OUT, C=C)

    y, slp = pl.pallas_call(
        body,
        grid=(_NCORES, T),
        in_specs=[
            pl.BlockSpec((1, BC, IN), lambda i, t: (t, i, 0)),
            pl.BlockSpec(W_x.shape, lambda i, t: (0, 0)),
            pl.BlockSpec(W_r.shape, lambda i, t: (0, 0)),
            pl.BlockSpec(W_hh.shape, lambda i, t: (0, 0)),
            pl.BlockSpec(bl.shape, lambda i, t: (0, 0)),
            pl.BlockSpec(W_post.shape, lambda i, t: (0, 0)),
            pl.BlockSpec(b_post.shape, lambda i, t: (0, 0)),
        ],
        out_specs=[
            pl.BlockSpec((1, BC, OUT), lambda i, t: (t, i, 0)),
            pl.BlockSpec((1, 8, 128), lambda i, t: (i, 0, 0)),
        ],
        out_shape=[
            jax.ShapeDtypeStruct((T, B, OUT), jnp.float32),
            jax.ShapeDtypeStruct((_NCORES, 8, 128), jnp.float32),
        ],
        scratch_shapes=[
            pltpu.VMEM((BC, C, N), jnp.float32),
            pltpu.VMEM((BC, H), jnp.float32),
            pltpu.VMEM((BC, H), jnp.float32),
            pltpu.VMEM((BC, 1, C), jnp.float32),
            pltpu.VMEM((BC, 1, N), jnp.float32),
            pltpu.VMEM((BC, C), jnp.float32),
            pltpu.VMEM((BC, 1, C), jnp.float32),
            pltpu.VMEM((BC, 1, 128), jnp.float32),
        ],
        compiler_params=pltpu.CompilerParams(
            dimension_semantics=("parallel", "arbitrary"),
            vmem_limit_bytes=56 * 1024 * 1024,
        ),
    )(xs, W_x, W_r, W_hh, bl, W_post, b_post)

    sup_loss = (slp[0, 0, 0] + slp[1, 0, 0]) / (B * C)
    return y, sup_loss


# speedup vs baseline: 1.5458x; 1.5458x over previous
import functools

import jax
import jax.numpy as jnp
from jax.experimental import pallas as pl
from jax.experimental.pallas import tpu as pltpu

_N = 2048
_EPS = 1e-8
_NCORES = 2


def _dnc_step(x_ref, wx_ref, wr_ref, whh_ref, bl_ref, wpost_ref, bpost_ref,
              y_ref, sl_ref,
              mem_ref, memb_ref, h_ref, c_ref, r_ref, sq_ref, slacc_ref,
              *, T, H, OUT, C):
    t = pl.program_id(1)

    @pl.when(t == 0)
    def _():
        mem_ref[...] = jnp.zeros_like(mem_ref)
        memb_ref[...] = jnp.zeros_like(memb_ref)
        h_ref[...] = jnp.zeros_like(h_ref)
        c_ref[...] = jnp.zeros_like(c_ref)
        r_ref[...] = jnp.zeros_like(r_ref)
        sq_ref[...] = jnp.zeros_like(sq_ref)
        slacc_ref[...] = jnp.zeros_like(slacc_ref)

    x = x_ref[0]
    h = h_ref[...]
    c = c_ref[...]
    BC = h.shape[0]
    r = r_ref[...]

    gates = (jnp.dot(x, wx_ref[...], preferred_element_type=jnp.float32)
             + jnp.dot(r, wr_ref[...], preferred_element_type=jnp.float32)
             + jnp.dot(h, whh_ref[...], preferred_element_type=jnp.float32)
             + bl_ref[...])
    i_g = gates[:, :H]
    f_g = gates[:, H:2 * H]
    g_g = gates[:, 2 * H:3 * H]
    o_g = gates[:, 3 * H:]
    c_new = jax.nn.sigmoid(f_g) * c + jax.nn.sigmoid(i_g) * jnp.tanh(g_g)
    h_new = jax.nn.sigmoid(o_g) * jnp.tanh(c_new)
    h_ref[...] = h_new
    c_ref[...] = c_new

    post = (jnp.dot(h_new, wpost_ref[...], preferred_element_type=jnp.float32)
            + bpost_ref[...])
    y_ref[0] = post[:, :OUT]
    key = post[:, OUT:OUT + C]
    gen = post[:, OUT + C:OUT + 2 * C]
    beta = jax.nn.softplus(post[:, OUT + 2 * C:OUT + 2 * C + 1])

    slacc_ref[...] += (key - gen) ** 2

    kk = jnp.sum(key * key, axis=1, keepdims=True)
    knorm = jnp.maximum(jnp.sqrt(kk), _EPS)

    K = BC * C
    key2 = jnp.concatenate([key, key], axis=1)
    ktiled = jnp.concatenate([key2] * (K // (2 * C)), axis=1)
    riota = jax.lax.broadcasted_iota(jnp.int32, (BC, K), 0)
    liota = jax.lax.broadcasted_iota(jnp.int32, (BC, K), 1)
    keybd = jnp.where((liota // C) == riota, ktiled, 0.0).astype(jnp.bfloat16)
    num = jnp.dot(keybd, memb_ref[...],
                  preferred_element_type=jnp.float32)

    mnorm = jnp.maximum(jnp.sqrt(jnp.maximum(sq_ref[...], 0.0)), _EPS)
    sim = num / (mnorm * knorm)
    wgt = jax.nn.softmax(beta * sim, axis=-1)

    mem = mem_ref[...]
    mem_new = mem + wgt[:, None, :] * key[:, :, None]
    mem_ref[...] = mem_new
    memb_ref[...] = mem_new.reshape(K, memb_ref.shape[-1]).astype(jnp.bfloat16)
    r_ref[...] = jnp.sum(mem_new * wgt[:, None, :], axis=2)
    sq_ref[...] = sq_ref[...] + 2.0 * wgt * num + (wgt * wgt) * kk

    @pl.when(t == T - 1)
    def _():
        sl_ref[0] = jnp.full((8, 128), jnp.sum(slacc_ref[...]), jnp.float32)


def kernel(xs, W_ih, W_hh, b_lstm, W_out, b_out, W_key, b_key,
           W_beta, b_beta, W_gen, b_gen):
    T, B, IN = xs.shape
    H = W_hh.shape[0]
    OUT = W_out.shape[1]
    C = W_key.shape[1]
    R = (W_ih.shape[0] - IN) // C
    N = _N
    BC = B // _NCORES

    W_x = W_ih[:IN]
    W_r = W_ih[IN:].reshape(C, R, 4 * H).sum(axis=1)
    post_w = jnp.concatenate([W_out, W_key, W_gen, W_beta], axis=1)
    post_pad = (-post_w.shape[1]) % 128
    W_post = jnp.pad(post_w, ((0, 0), (0, post_pad)))
    b_post = jnp.pad(jnp.concatenate([b_out, b_key, b_gen, b_beta]),
                     (0, post_pad))[None]
    bl = b_lstm[None]

    body = functools.partial(_dnc_step, T=T, H=H, OUT=OUT, C=C)

    y, slp = pl.pallas_call(
        body,
        grid=(_NCORES, T),
        in_specs=[
            pl.BlockSpec((1, BC, IN), lambda i, t: (t, i, 0)),
            pl.BlockSpec(W_x.shape, lambda i, t: (0, 0)),
            pl.BlockSpec(W_r.shape, lambda i, t: (0, 0)),
            pl.BlockSpec(W_hh.shape, lambda i, t: (0, 0)),
            pl.BlockSpec(bl.shape, lambda i, t: (0, 0)),
            pl.BlockSpec(W_post.shape, lambda i, t: (0, 0)),
            pl.BlockSpec(b_post.shape, lambda i, t: (0, 0)),
        ],
        out_specs=[
            pl.BlockSpec((1, BC, OUT), lambda i, t: (t, i, 0)),
            pl.BlockSpec((1, 8, 128), lambda i, t: (i, 0, 0)),
        ],
        out_shape=[
            jax.ShapeDtypeStruct((T, B, OUT), jnp.float32),
            jax.ShapeDtypeStruct((_NCORES, 8, 128), jnp.float32),
        ],
        scratch_shapes=[
            pltpu.VMEM((BC, C, N), jnp.float32),
            pltpu.VMEM((BC * C, N), jnp.bfloat16),
            pltpu.VMEM((BC, H), jnp.float32),
            pltpu.VMEM((BC, H), jnp.float32),
            pltpu.VMEM((BC, C), jnp.float32),
            pltpu.VMEM((BC, N), jnp.float32),
            pltpu.VMEM((BC, C), jnp.float32),
        ],
        compiler_params=pltpu.CompilerParams(
            dimension_semantics=("parallel", "arbitrary"),
            vmem_limit_bytes=56 * 1024 * 1024,
        ),
    )(xs, W_x, W_r, W_hh, bl, W_post, b_post)

    sup_loss = (slp[0, 0, 0] + slp[1, 0, 0]) / (B * C)
    return y, sup_loss
